# baseline (speedup 1.0000x reference)
import math

import jax
import jax.numpy as jnp
from jax.experimental import pallas as pl
from jax.experimental.pallas import tpu as pltpu

_SIZE = 32000
_PAD = 0
_SMOOTH = 0.1
_CONF = 1.0 - _SMOOTH
_FILL = _SMOOTH / (_SIZE - 2)
_C0 = _CONF * math.log(_CONF) + _SMOOTH * math.log(_FILL)
_K = _CONF / _FILL

_ROWS = 256


def _body(t_ref, x_ref, out_ref):
    j = pl.program_id(0)
    x = x_ref[...]
    r, size = x.shape
    t = t_ref[:, 0]

    p = jnp.zeros((r, 128), jnp.float32)
    for k in range(size // 128):
        xs = x[:, k * 128:(k + 1) * 128]
        cid = k * 128 + jax.lax.broadcasted_iota(jnp.int32, (r, 128), 1)
        z = jnp.where(cid == t[:, None], _K * xs, xs)
        if k == 0:
            z = jnp.where(cid == 0, 0.0, z)
        p = p + z

    ones = jnp.ones((128, 1), jnp.float32)
    rowz = jax.lax.dot(p, ones, preferred_element_type=jnp.float32)[:, 0]
    validf = (t != _PAD).astype(jnp.float32)
    partial = jnp.sum(validf) * _C0 - _FILL * jnp.sum(validf * rowz)

    @pl.when(j == 0)
    def _():
        out_ref[...] = partial.reshape(1, 1)

    @pl.when(j > 0)
    def _():
        out_ref[...] += partial.reshape(1, 1)


@jax.jit
def kernel(x, target):
    n, size = x.shape
    t2 = target.reshape(n, 1)
    grid = n // _ROWS
    out = pl.pallas_call(
        _body,
        grid=(grid,),
        in_specs=[
            pl.BlockSpec((_ROWS, 1), lambda j: (j, 0)),
            pl.BlockSpec((_ROWS, size), lambda j: (j, 0)),
        ],
        out_specs=pl.BlockSpec((1, 1), lambda j: (0, 0)),
        out_shape=jax.ShapeDtypeStruct((1, 1), jnp.float32),
        compiler_params=pltpu.CompilerParams(vmem_limit_bytes=100 * 1024 * 1024),
    )(t2, x)
    return out[0, 0]


# speedup vs baseline: 1.0597x; 1.0597x over previous
import math

import jax
import jax.numpy as jnp
from jax.experimental import pallas as pl
from jax.experimental.pallas import tpu as pltpu

_SIZE = 32000
_PAD = 0
_SMOOTH = 0.1
_CONF = 1.0 - _SMOOTH
_FILL = _SMOOTH / (_SIZE - 2)
_C0 = _CONF * math.log(_CONF) + _SMOOTH * math.log(_FILL)
_K = _CONF / _FILL

_ROWS = 64


def _half(x, t):
    r, size = x.shape
    p = jnp.zeros((r, 128), jnp.float32)
    for k in range(size // 128):
        xs = x[:, k * 128:(k + 1) * 128]
        cid = k * 128 + jax.lax.broadcasted_iota(jnp.int32, (r, 128), 1)
        z = jnp.where(cid == t[:, None], _K * xs, xs)
        if k == 0:
            z = jnp.where(cid == 0, 0.0, z)
        p = p + z

    ones = jnp.ones((128, 1), jnp.float32)
    rowz = jax.lax.dot(p, ones, preferred_element_type=jnp.float32)[:, 0]
    validf = (t != _PAD).astype(jnp.float32)
    return jnp.sum(validf) * _C0 - _FILL * jnp.sum(validf * rowz)


def _body(ta_ref, tb_ref, xa_ref, xb_ref, out_ref):
    j = pl.program_id(0)
    partial = (_half(xa_ref[...], ta_ref[:, 0])
               + _half(xb_ref[...], tb_ref[:, 0]))

    @pl.when(j == 0)
    def _():
        out_ref[...] = partial.reshape(1, 1)

    @pl.when(j > 0)
    def _():
        out_ref[...] += partial.reshape(1, 1)


@jax.jit
def kernel(x, target):
    n, size = x.shape
    t2 = target.reshape(n, 1)
    grid = n // _ROWS // 2
    out = pl.pallas_call(
        _body,
        grid=(grid,),
        in_specs=[
            pl.BlockSpec((_ROWS, 1), lambda j: (j, 0)),
            pl.BlockSpec((_ROWS, 1), lambda j, g=grid: (j + g, 0)),
            pl.BlockSpec((_ROWS, size), lambda j: (j, 0)),
            pl.BlockSpec((_ROWS, size), lambda j, g=grid: (j + g, 0)),
        ],
        out_specs=pl.BlockSpec((1, 1), lambda j: (0, 0)),
        out_shape=jax.ShapeDtypeStruct((1, 1), jnp.float32),
    )(t2, t2, x, x)
    return out[0, 0]
